# baseline (speedup 1.0000x reference)
import functools

import jax
import jax.numpy as jnp
from jax import lax
from jax.experimental import pallas as pl
from jax.experimental.pallas import tpu as pltpu
from jax.experimental.pallas import tpu_sc as plsc

_N_ROWS = 4096
_N_COLS = 50
_B = _N_ROWS * _N_COLS
_D = 128
_NC = 2
_NS = 16
_NW = _NC * _NS
_BPW = _B // _NW
_CH = 400
_NCHUNK = _BPW // _CH

_mesh = plsc.VectorSubcoreMesh(core_axis_name="c", subcore_axis_name="s")


@functools.partial(
    pl.kernel,
    mesh=_mesh,
    out_type=jax.ShapeDtypeStruct((_B, _D), jnp.float32),
    scratch_types=[
        pltpu.VMEM((_CH,), jnp.int32),
        pltpu.VMEM((_CH, _D), jnp.float32),
        pltpu.SemaphoreType.DMA,
    ],
)
def _embed_sc(cue_hbm, table_hbm, out_hbm, idx_v, rows_v, sem):
    wid = lax.axis_index("s") * _NC + lax.axis_index("c")
    base = wid * _BPW

    def body(i, carry):
        off = base + i * _CH
        pltpu.sync_copy(cue_hbm.at[pl.ds(off, _CH)], idx_v)
        pltpu.async_copy(table_hbm.at[idx_v], rows_v, sem).wait()
        pltpu.sync_copy(rows_v, out_hbm.at[pl.ds(off, _CH)])
        return carry

    lax.fori_loop(0, _NCHUNK, body, 0)


def kernel(cue, table):
    idx = cue.reshape(_B).astype(jnp.int32)
    out = _embed_sc(idx, table.astype(jnp.float32))
    return out.reshape(_N_ROWS, _N_COLS, _D)


# speedup vs baseline: 4.6254x; 4.6254x over previous
import functools

import jax
import jax.numpy as jnp
from jax import lax
from jax.experimental import pallas as pl
from jax.experimental.pallas import tpu as pltpu
from jax.experimental.pallas import tpu_sc as plsc

_N_ROWS = 4096
_N_COLS = 50
_B = _N_ROWS * _N_COLS
_D = 128
_NC = 2
_NS = 16
_NW = _NC * _NS
_BPW = _B // _NW
_CH = 400
_NCHUNK = _BPW // _CH

_mesh = plsc.VectorSubcoreMesh(core_axis_name="c", subcore_axis_name="s")


@functools.partial(
    pl.kernel,
    mesh=_mesh,
    out_type=jax.ShapeDtypeStruct((_B, _D), jnp.float32),
    scratch_types=[
        pltpu.VMEM((_BPW,), jnp.int32),
        pltpu.VMEM((2, _CH, _D), jnp.float32),
        pltpu.SemaphoreType.DMA,
        pltpu.SemaphoreType.DMA,
        pltpu.SemaphoreType.DMA,
    ],
)
def _embed_sc(cue_hbm, table_hbm, out_hbm, idx_v, rows_v, gsem, osem0, osem1):
    wid = lax.axis_index("s") * _NC + lax.axis_index("c")
    base = wid * _BPW
    pltpu.sync_copy(cue_hbm.at[pl.ds(base, _BPW)], idx_v)

    osems = (osem0, osem1)
    pending = [None, None]
    for c in range(_NCHUNK):
        b = c % 2
        if pending[b] is not None:
            pending[b].wait()
        pltpu.async_copy(
            table_hbm.at[idx_v.at[pl.ds(c * _CH, _CH)]], rows_v.at[b], gsem
        ).wait()
        pending[b] = pltpu.async_copy(
            rows_v.at[b], out_hbm.at[pl.ds(base + c * _CH, _CH)], osems[b]
        )
    for b in range(2):
        if pending[b] is not None:
            pending[b].wait()


def kernel(cue, table):
    idx = cue.reshape(_B).astype(jnp.int32)
    idx = idx + 4 * (jnp.arange(_B, dtype=jnp.int32) // _BPW)
    table_rep = jnp.tile(table.astype(jnp.float32), (_NW, 1))
    out = _embed_sc(idx, table_rep)
    return out.reshape(_N_ROWS, _N_COLS, _D)


# speedup vs baseline: 6.1611x; 1.3320x over previous
import functools

import jax
import jax.numpy as jnp
from jax import lax
from jax.experimental import pallas as pl
from jax.experimental.pallas import tpu as pltpu
from jax.experimental.pallas import tpu_sc as plsc

_N_ROWS = 4096
_N_COLS = 50
_B = _N_ROWS * _N_COLS
_D = 128
_DQ = 4 * _D
_BQ = _B // 4
_NC = 2
_NS = 16
_NW = _NC * _NS
_BPW = _BQ // _NW
_CH = 80
_NCHUNK = _BPW // _CH
_NBUF = 3

_mesh = plsc.VectorSubcoreMesh(core_axis_name="c", subcore_axis_name="s")


@functools.partial(
    pl.kernel,
    mesh=_mesh,
    out_type=jax.ShapeDtypeStruct((_BQ, _DQ), jnp.float32),
    scratch_types=[
        pltpu.VMEM((_BPW,), jnp.int32),
        pltpu.VMEM((_NBUF, _CH, _DQ), jnp.float32),
        pltpu.SemaphoreType.DMA,
        pltpu.SemaphoreType.DMA,
        pltpu.SemaphoreType.DMA,
        pltpu.SemaphoreType.DMA,
        pltpu.SemaphoreType.DMA,
        pltpu.SemaphoreType.DMA,
    ],
)
def _embed_sc(qidx_hbm, qtable_hbm, out_hbm, idx_v, rows_v,
              g0, g1, g2, o0, o1, o2):
    wid = lax.axis_index("s") * _NC + lax.axis_index("c")
    base = wid * _BPW
    pltpu.sync_copy(qidx_hbm.at[pl.ds(base, _BPW)], idx_v)

    gsems = (g0, g1, g2)
    osems = (o0, o1, o2)

    def start_gather(c):
        b = c % _NBUF
        return pltpu.async_copy(
            qtable_hbm.at[idx_v.at[pl.ds(c * _CH, _CH)]], rows_v.at[b],
            gsems[b])

    gather_pending = [None] * _NBUF
    out_pending = [None] * _NBUF
    for c in range(min(2, _NCHUNK)):
        gather_pending[c % _NBUF] = start_gather(c)
    for c in range(_NCHUNK):
        b = c % _NBUF
        gather_pending[b].wait()
        out_pending[b] = pltpu.async_copy(
            rows_v.at[b], out_hbm.at[pl.ds(base + c * _CH, _CH)], osems[b])
        n = c + 2
        if n < _NCHUNK:
            bn = n % _NBUF
            if out_pending[bn] is not None:
                out_pending[bn].wait()
                out_pending[bn] = None
            gather_pending[bn] = start_gather(n)
    for b in range(_NBUF):
        if out_pending[b] is not None:
            out_pending[b].wait()


def kernel(cue, table):
    idx = cue.reshape(_BQ, 4).astype(jnp.int32)
    qidx = ((idx[:, 0] * 4 + idx[:, 1]) * 4 + idx[:, 2]) * 4 + idx[:, 3]
    digits = (jnp.arange(256, dtype=jnp.int32)[:, None]
              // jnp.array([64, 16, 4, 1], dtype=jnp.int32)) % 4
    qtable = jnp.take(table.astype(jnp.float32), digits, axis=0)
    qtable = qtable.reshape(256, _DQ)
    out = _embed_sc(qidx, qtable)
    return out.reshape(_N_ROWS, _N_COLS, _D)
